# baseline (speedup 1.0000x reference)
import jax
import jax.numpy as jnp
import numpy as np
from jax import lax
from jax.experimental import pallas as pl
from jax.experimental.pallas import tpu as pltpu
from jax.experimental.pallas import tpu_sc as plsc

_B = 16
_NC, _NS = 2, 16
_NW = _NC * _NS


def _fixed_indices(seed, shape, num_remain):
    with jax.default_device(jax.devices("cpu")[0]):
        noise = jax.random.uniform(jax.random.key(seed), shape)
        shuffle_idx = jnp.argsort(noise, axis=-1)
        remain = np.asarray(shuffle_idx[..., :num_remain], dtype=np.int32)
        masked = np.asarray(shuffle_idx[..., num_remain:], dtype=np.int32)
        revert = np.asarray(jnp.argsort(shuffle_idx, axis=-1), dtype=np.int32)
    return remain, masked, revert


_T_REMAIN, _T_MASKED, _T_REVERT = _fixed_indices(1, (_B, 8, 512), 256)
_I_REMAIN, _I_MASKED, _I_REVERT = _fixed_indices(2, (_B, 576), 144)

_NLP_ROWS, _NLP_N8, _NLP_FULL = 16 * 513, 1026, 256
_IMG_ROWS, _IMG_N8, _IMG_FULL = 16 * 145, 290, 72
_TMP_ROWS, _TMP_N8, _TMP_FULL = 128 * 257, 4112, 1024

_S768, _S256 = 32, 64

_NLP_CHUNKS = [(i * 32, 32) for i in range(8)]
_IMG_CHUNKS = [(0, 32), (32, 32), (64, 8)]
_TMP_CHUNKS = [(i * 64, 64) for i in range(16)]


def _flat_src(remain, rows_per_seq):
    lead = remain.reshape(-1, remain.shape[-1]).astype(np.int32)
    n_seq = lead.shape[0]
    src = np.concatenate([np.zeros((n_seq, 1), np.int32), lead + 1], axis=1)
    src += (np.arange(n_seq, dtype=np.int32) * rows_per_seq)[:, None]
    return src.reshape(-1)


_SRC_IMG = _flat_src(_I_REMAIN, 577)
_SRC_TMP = _flat_src(_T_REMAIN, 513)


def _nlp_src(nlp_remain_idx):
    b = nlp_remain_idx.shape[0]
    src = jnp.concatenate(
        [jnp.zeros((b, 1), jnp.int32), nlp_remain_idx.astype(jnp.int32) + 1], axis=1
    )
    src = src + (jnp.arange(b, dtype=jnp.int32) * 2049)[:, None]
    return src.reshape(-1)


class _Ring:
    def __init__(self, bufs, gsems, wsems):
        self.bufs, self.gsems, self.wsems = bufs, gsems, wsems
        self.gh = [None] * len(bufs)
        self.wh = [None] * len(bufs)
        self.last = None
        self.ptr = 0

    def issue(self, hbm, idxbuf, off, out_ref, out_base, rows):
        s = self.ptr % len(self.bufs)
        self.ptr += 1
        if self.wh[s] is not None:
            self.wh[s].wait()
            self.wh[s] = None
        self.gh[s] = pltpu.async_copy(
            hbm.at[idxbuf.at[pl.ds(off, rows)]],
            self.bufs[s].at[pl.ds(0, rows)],
            self.gsems[s],
        )
        if self.last is not None:
            ls, lout, lbase, lrows = self.last
            self.gh[ls].wait()
            self.gh[ls] = None
            self.wh[ls] = pltpu.async_copy(
                self.bufs[ls].at[pl.ds(0, lrows)],
                lout.at[pl.ds(lbase, lrows)],
                self.wsems[ls],
            )
        self.last = (s, out_ref, out_base, rows)

    def drain(self):
        if self.last is not None:
            ls, lout, lbase, lrows = self.last
            self.gh[ls].wait()
            self.wh[ls] = pltpu.async_copy(
                self.bufs[ls].at[pl.ds(0, lrows)],
                lout.at[pl.ds(lbase, lrows)],
                self.wsems[ls],
            )
            self.last = None
        for s, h in enumerate(self.wh):
            if h is not None:
                h.wait()
                self.wh[s] = None


def _share(wid, n8):
    b = ((wid * n8) // 32) * 8
    e = (((wid + 1) * n8) // 32) * 8
    return b, e


def _gather_body(nlp_hbm, img_hbm, tmp_hbm, src_nlp, src_img, src_tmp,
                 out_nlp, out_img, out_tmp,
                 idx_nlp, idx_img, idx_tmp,
                 ti_nlp, ti_img, ti_tmp,
                 d768a, d768b, d768c, d256a, d256b, d256c,
                 g768a, g768b, g768c, w768a, w768b, w768c,
                 g256a, g256b, g256c, w256a, w256b, w256c,
                 tg_a, tg_b, tg_c, tw_a, tw_b, tw_c):
    wid = lax.axis_index("s") * _NC + lax.axis_index("c")
    b_nlp, e_nlp = _share(wid, _NLP_N8)
    b_img, e_img = _share(wid, _IMG_N8)
    b_tmp, e_tmp = _share(wid, _TMP_N8)

    pltpu.sync_copy(src_img.at[pl.ds(b_img, _IMG_FULL)], idx_img)
    pltpu.sync_copy(src_nlp.at[pl.ds(b_nlp, _NLP_FULL)], idx_nlp)
    pltpu.sync_copy(src_tmp.at[pl.ds(b_tmp, _TMP_FULL)], idx_tmp)

    r768 = _Ring([d768a, d768b, d768c], [g768a, g768b, g768c],
                 [w768a, w768b, w768c])
    r256 = _Ring([d256a, d256b, d256c], [g256a, g256b, g256c],
                 [w256a, w256b, w256c])

    t768 = ([(r768, img_hbm, idx_img, off, out_img, b_img + off, rows)
             for off, rows in _IMG_CHUNKS] +
            [(r768, nlp_hbm, idx_nlp, off, out_nlp, b_nlp + off, rows)
             for off, rows in _NLP_CHUNKS])
    t256 = [(r256, tmp_hbm, idx_tmp, off, out_tmp, b_tmp + off, rows)
            for off, rows in _TMP_CHUNKS]

    merged = []
    n = max(len(t768), len(t256))
    for i in range(n):
        if i < len(t256):
            merged.append(t256[i])
        if i < len(t768):
            merged.append(t768[i])
    for ring, hbm, idxbuf, off, out_ref, out_base, rows in merged:
        ring.issue(hbm, idxbuf, off, out_ref, out_base, rows)

    tails = [
        (nlp_hbm, src_nlp, out_nlp, ti_nlp,
         jnp.where(e_nlp - b_nlp > _NLP_FULL, b_nlp + _NLP_FULL,
                   b_nlp + _NLP_FULL - 8),
         d768a, tg_a, tw_a),
        (img_hbm, src_img, out_img, ti_img,
         jnp.where(e_img - b_img > _IMG_FULL, b_img + _IMG_FULL,
                   b_img + _IMG_FULL - 8),
         d768b, tg_b, tw_b),
        (tmp_hbm, src_tmp, out_tmp, ti_tmp,
         jnp.where(e_tmp - b_tmp > _TMP_FULL, b_tmp + _TMP_FULL,
                   b_tmp + _TMP_FULL - 8),
         d256a, tg_c, tw_c),
    ]
    r768.drain()
    r256.drain()
    gh = []
    for hbm, src, out_ref, tibuf, gbase, dbuf, gsem, wsem in tails:
        pltpu.sync_copy(src.at[pl.ds(gbase, 8)], tibuf)
        gh.append(pltpu.async_copy(hbm.at[tibuf], dbuf.at[pl.ds(0, 8)], gsem))
    wh = []
    for (hbm, src, out_ref, tibuf, gbase, dbuf, gsem, wsem), h in zip(tails, gh):
        h.wait()
        wh.append(pltpu.async_copy(dbuf.at[pl.ds(0, 8)],
                                   out_ref.at[pl.ds(gbase, 8)], wsem))
    for h in wh:
        h.wait()


_gather_call = pl.kernel(
    _gather_body,
    out_type=(
        jax.ShapeDtypeStruct((_NLP_ROWS, 768), jnp.float32),
        jax.ShapeDtypeStruct((_IMG_ROWS, 768), jnp.float32),
        jax.ShapeDtypeStruct((_TMP_ROWS, 256), jnp.float32),
    ),
    mesh=plsc.VectorSubcoreMesh(core_axis_name="c", subcore_axis_name="s"),
    scratch_types=(
        pltpu.VMEM((_NLP_FULL,), jnp.int32),
        pltpu.VMEM((_IMG_FULL,), jnp.int32),
        pltpu.VMEM((_TMP_FULL,), jnp.int32),
        pltpu.VMEM((8,), jnp.int32),
        pltpu.VMEM((8,), jnp.int32),
        pltpu.VMEM((8,), jnp.int32),
        pltpu.VMEM((_S768, 768), jnp.float32),
        pltpu.VMEM((_S768, 768), jnp.float32),
        pltpu.VMEM((_S768, 768), jnp.float32),
        pltpu.VMEM((_S256, 256), jnp.float32),
        pltpu.VMEM((_S256, 256), jnp.float32),
        pltpu.VMEM((_S256, 256), jnp.float32),
    ) + (pltpu.SemaphoreType.DMA,) * 18,
)


def kernel(temporal_block, img, nlp, nlp_remain_idx, nlp_masked_idx,
           nlp_revert_idx, nlp_revert_padding_mask):
    z = lax.optimization_barrier(jnp.zeros((), jnp.float32))
    nlp_flat = (nlp + z).reshape(-1, nlp.shape[-1])
    img_flat = (img + z).reshape(-1, img.shape[-1])
    tmp_flat = (temporal_block + z).reshape(-1, temporal_block.shape[-1])
    src_nlp = _nlp_src(nlp_remain_idx)

    out_nlp_p, out_img_p, out_tmp_p = _gather_call(
        nlp_flat, img_flat, tmp_flat,
        src_nlp, jnp.asarray(_SRC_IMG), jnp.asarray(_SRC_TMP),
    )

    temporal_remain_block = (out_tmp_p + z).reshape(_B, 8, 257, 256)
    img_remain = (out_img_p + z).reshape(_B, 145, 768)
    nlp_remain = (out_nlp_p + z).reshape(_B, 513, 768)

    ng_pm = nlp_revert_padding_mask[:, :1]
    nv_pm = nlp_revert_padding_mask[:, 1:]
    nr_pm = jnp.take_along_axis(nv_pm, nlp_remain_idx, axis=1)
    nm_pm = jnp.take_along_axis(nv_pm, nlp_masked_idx, axis=1)
    nlp_remain_pm = jnp.concatenate([ng_pm, nr_pm], axis=1)
    nlp_masked_pm = jnp.concatenate([ng_pm, nm_pm], axis=1)
    img_remain_pm = jnp.ones((_B, 145), jnp.float32)
    img_masked_pm = jnp.ones((_B, 433), jnp.float32)
    img_revert_pm = jnp.ones((_B, 577), jnp.float32)

    return (temporal_remain_block, jnp.asarray(_T_MASKED), jnp.asarray(_T_REVERT),
            img_remain, jnp.asarray(_I_MASKED), jnp.asarray(_I_REVERT),
            img_remain_pm, img_masked_pm, img_revert_pm,
            nlp_remain, nlp_remain_pm, nlp_masked_pm, nlp_revert_padding_mask)


# speedup vs baseline: 1.0633x; 1.0633x over previous
import jax
import jax.numpy as jnp
import numpy as np
from jax import lax
from jax.experimental import pallas as pl
from jax.experimental.pallas import tpu as pltpu
from jax.experimental.pallas import tpu_sc as plsc

_B = 16
_NC, _NS = 2, 16
_NW = _NC * _NS


def _fixed_indices(seed, shape, num_remain):
    with jax.default_device(jax.devices("cpu")[0]):
        noise = jax.random.uniform(jax.random.key(seed), shape)
        shuffle_idx = jnp.argsort(noise, axis=-1)
        remain = np.asarray(shuffle_idx[..., :num_remain], dtype=np.int32)
        masked = np.asarray(shuffle_idx[..., num_remain:], dtype=np.int32)
        revert = np.asarray(jnp.argsort(shuffle_idx, axis=-1), dtype=np.int32)
    return remain, masked, revert


_T_REMAIN, _T_MASKED, _T_REVERT = _fixed_indices(1, (_B, 8, 512), 256)
_I_REMAIN, _I_MASKED, _I_REVERT = _fixed_indices(2, (_B, 576), 144)


_NLP_IN_S, _IMG_IN_S, _TMP_IN_S = 2056, 584, 520
_NLP_OUT_R, _IMG_OUT_R, _TMP_OUT_R = 513, 145, 257
_NLP_OUT_S, _IMG_OUT_S, _TMP_OUT_S = 520, 152, 264

_NLP_CHUNKS = [(i * 32, 32) for i in range(8)] + [(256, 8)]
_IMG_CHUNKS = [(0, 32), (32, 32), (64, 16)]
_TMP_CHUNKS = [(i * 64, 64) for i in range(4)] + [(256, 8)]

_S768, _S256 = 32, 64


def _flat_src(remain, in_stride, out_stride):
    lead = remain.reshape(-1, remain.shape[-1]).astype(np.int32)
    n_seq = lead.shape[0]
    src = np.concatenate([np.zeros((n_seq, 1), np.int32), lead + 1], axis=1)
    src += (np.arange(n_seq, dtype=np.int32) * in_stride)[:, None]
    out = np.zeros((n_seq, out_stride), np.int32)
    out[:, : src.shape[1]] = src
    return out.reshape(-1)


_SRC_IMG = _flat_src(_I_REMAIN, _IMG_IN_S, _IMG_OUT_S)
_SRC_TMP = _flat_src(_T_REMAIN, _TMP_IN_S, _TMP_OUT_S)


def _nlp_src(nlp_remain_idx):
    b = nlp_remain_idx.shape[0]
    src = jnp.concatenate(
        [jnp.zeros((b, 1), jnp.int32), nlp_remain_idx.astype(jnp.int32) + 1], axis=1
    )
    src = src + (jnp.arange(b, dtype=jnp.int32) * _NLP_IN_S)[:, None]
    src = jnp.pad(src, ((0, 0), (0, _NLP_OUT_S - src.shape[1])))
    return src.reshape(-1)


def _pad_rows(x, stride):
    b, r, w = x.shape

    def body(x_ref, o_ref):
        o_ref[pl.ds(0, r), :] = x_ref[0]

    return pl.pallas_call(
        body,
        grid=(b,),
        in_specs=[pl.BlockSpec((1, r, w), lambda i: (i, 0, 0))],
        out_specs=pl.BlockSpec((stride, w), lambda i: (i, 0)),
        out_shape=jax.ShapeDtypeStruct((b * stride, w), x.dtype),
    )(x)


def _unpad_rows(x, b, stride, r_out):
    w = x.shape[-1]

    def body(x_ref, o_ref):
        o_ref[0] = x_ref[pl.ds(0, r_out), :]

    return pl.pallas_call(
        body,
        grid=(b,),
        in_specs=[pl.BlockSpec((stride, w), lambda i: (i, 0))],
        out_specs=pl.BlockSpec((1, r_out, w), lambda i: (i, 0, 0)),
        out_shape=jax.ShapeDtypeStruct((b, r_out, w), x.dtype),
    )(x)


class _Ring:
    def __init__(self, bufs, gsems, wsems):
        self.bufs, self.gsems, self.wsems = bufs, gsems, wsems
        self.gh = [None] * len(bufs)
        self.wh = [None] * len(bufs)
        self.last = None
        self.ptr = 0

    def issue(self, hbm, idxbuf, off, out_ref, out_base, rows):
        s = self.ptr % len(self.bufs)
        self.ptr += 1
        if self.wh[s] is not None:
            self.wh[s].wait()
            self.wh[s] = None
        self.gh[s] = pltpu.async_copy(
            hbm.at[idxbuf.at[pl.ds(off, rows)]],
            self.bufs[s].at[pl.ds(0, rows)],
            self.gsems[s],
        )
        if self.last is not None:
            ls, lout, lbase, lrows = self.last
            self.gh[ls].wait()
            self.gh[ls] = None
            self.wh[ls] = pltpu.async_copy(
                self.bufs[ls].at[pl.ds(0, lrows)],
                lout.at[pl.ds(lbase, lrows)],
                self.wsems[ls],
            )
        self.last = (s, out_ref, out_base, rows)

    def drain(self):
        if self.last is not None:
            ls, lout, lbase, lrows = self.last
            self.gh[ls].wait()
            self.wh[ls] = pltpu.async_copy(
                self.bufs[ls].at[pl.ds(0, lrows)],
                lout.at[pl.ds(lbase, lrows)],
                self.wsems[ls],
            )
            self.last = None
        for s, h in enumerate(self.wh):
            if h is not None:
                h.wait()
                self.wh[s] = None


def _gather_body(nlp_hbm, img_hbm, tmp_hbm, src_nlp, src_img, src_tmp,
                 out_nlp, out_img, out_tmp,
                 idx_nlp, idx_img, idx_tmp,
                 d768a, d768b, d768c, d256a, d256b, d256c,
                 g768a, g768b, g768c, w768a, w768b, w768c,
                 g256a, g256b, g256c, w256a, w256b, w256c):
    wid = lax.axis_index("s") * _NC + lax.axis_index("c")
    seq = wid // 2
    half = wid % 2

    pltpu.sync_copy(src_img.at[pl.ds(seq * _IMG_OUT_S + half * 72, 80)], idx_img)
    pltpu.sync_copy(src_nlp.at[pl.ds(seq * _NLP_OUT_S + half * 256, 264)], idx_nlp)
    pltpu.sync_copy(src_tmp.at[pl.ds(wid * (4 * _TMP_OUT_S), 1056)], idx_tmp)

    r768 = _Ring([d768a, d768b, d768c], [g768a, g768b, g768c],
                 [w768a, w768b, w768c])
    r256 = _Ring([d256a, d256b, d256c], [g256a, g256b, g256c],
                 [w256a, w256b, w256c])

    img_base = seq * _IMG_OUT_S + half * 72
    nlp_base = seq * _NLP_OUT_S + half * 256
    t768 = ([(r768, img_hbm, idx_img, off, out_img, img_base + off, rows)
             for off, rows in _IMG_CHUNKS] +
            [(r768, nlp_hbm, idx_nlp, off, out_nlp, nlp_base + off, rows)
             for off, rows in _NLP_CHUNKS])
    t256 = []
    for j in range(4):
        slab = wid * 4 + j
        for off, rows in _TMP_CHUNKS:
            t256.append((r256, tmp_hbm, idx_tmp, j * _TMP_OUT_S + off,
                         out_tmp, slab * _TMP_OUT_S + off, rows))

    merged = []
    n = max(len(t768), len(t256))
    for i in range(n):
        if i < len(t256):
            merged.append(t256[i])
        if i < len(t768):
            merged.append(t768[i])
    for ring, hbm, idxbuf, off, out_ref, out_base, rows in merged:
        ring.issue(hbm, idxbuf, off, out_ref, out_base, rows)
    r768.drain()
    r256.drain()


_gather_call = pl.kernel(
    _gather_body,
    out_type=(
        jax.ShapeDtypeStruct((_B * _NLP_OUT_S, 768), jnp.float32),
        jax.ShapeDtypeStruct((_B * _IMG_OUT_S, 768), jnp.float32),
        jax.ShapeDtypeStruct((128 * _TMP_OUT_S, 256), jnp.float32),
    ),
    mesh=plsc.VectorSubcoreMesh(core_axis_name="c", subcore_axis_name="s"),
    scratch_types=(
        pltpu.VMEM((264,), jnp.int32),
        pltpu.VMEM((80,), jnp.int32),
        pltpu.VMEM((1056,), jnp.int32),
        pltpu.VMEM((_S768, 768), jnp.float32),
        pltpu.VMEM((_S768, 768), jnp.float32),
        pltpu.VMEM((_S768, 768), jnp.float32),
        pltpu.VMEM((_S256, 256), jnp.float32),
        pltpu.VMEM((_S256, 256), jnp.float32),
        pltpu.VMEM((_S256, 256), jnp.float32),
    ) + (pltpu.SemaphoreType.DMA,) * 12,
)


def kernel(temporal_block, img, nlp, nlp_remain_idx, nlp_masked_idx,
           nlp_revert_idx, nlp_revert_padding_mask):
    nlp_stage = _pad_rows(nlp, _NLP_IN_S)
    img_stage = _pad_rows(img, _IMG_IN_S)
    tmp_stage = _pad_rows(temporal_block.reshape(128, 513, 256), _TMP_IN_S)
    src_nlp = _nlp_src(nlp_remain_idx)

    out_nlp_p, out_img_p, out_tmp_p = _gather_call(
        nlp_stage, img_stage, tmp_stage,
        src_nlp, jnp.asarray(_SRC_IMG), jnp.asarray(_SRC_TMP),
    )

    nlp_remain = _unpad_rows(out_nlp_p, _B, _NLP_OUT_S, _NLP_OUT_R)
    img_remain = _unpad_rows(out_img_p, _B, _IMG_OUT_S, _IMG_OUT_R)
    temporal_remain_block = _unpad_rows(
        out_tmp_p, 128, _TMP_OUT_S, _TMP_OUT_R).reshape(_B, 8, 257, 256)

    ng_pm = nlp_revert_padding_mask[:, :1]
    nv_pm = nlp_revert_padding_mask[:, 1:]
    nr_pm = jnp.take_along_axis(nv_pm, nlp_remain_idx, axis=1)
    nm_pm = jnp.take_along_axis(nv_pm, nlp_masked_idx, axis=1)
    nlp_remain_pm = jnp.concatenate([ng_pm, nr_pm], axis=1)
    nlp_masked_pm = jnp.concatenate([ng_pm, nm_pm], axis=1)
    img_remain_pm = jnp.ones((_B, 145), jnp.float32)
    img_masked_pm = jnp.ones((_B, 433), jnp.float32)
    img_revert_pm = jnp.ones((_B, 577), jnp.float32)

    return (temporal_remain_block, jnp.asarray(_T_MASKED), jnp.asarray(_T_REVERT),
            img_remain, jnp.asarray(_I_MASKED), jnp.asarray(_I_REVERT),
            img_remain_pm, img_masked_pm, img_revert_pm,
            nlp_remain, nlp_remain_pm, nlp_masked_pm, nlp_revert_padding_mask)


# speedup vs baseline: 1.3482x; 1.2679x over previous
import jax
import jax.numpy as jnp
import numpy as np
from jax import lax
from jax.experimental import pallas as pl
from jax.experimental.pallas import tpu as pltpu
from jax.experimental.pallas import tpu_sc as plsc

_B = 16
_NC, _NS = 2, 16
_NW = _NC * _NS


def _fixed_indices(seed, shape, num_remain):
    with jax.default_device(jax.devices("cpu")[0]):
        noise = jax.random.uniform(jax.random.key(seed), shape)
        shuffle_idx = jnp.argsort(noise, axis=-1)
        remain = np.asarray(shuffle_idx[..., :num_remain], dtype=np.int32)
        masked = np.asarray(shuffle_idx[..., num_remain:], dtype=np.int32)
        revert = np.asarray(jnp.argsort(shuffle_idx, axis=-1), dtype=np.int32)
    return remain, masked, revert


_T_REMAIN, _T_MASKED, _T_REVERT = _fixed_indices(1, (_B, 8, 512), 256)
_I_REMAIN, _I_MASKED, _I_REVERT = _fixed_indices(2, (_B, 576), 144)

_NLP_ROWS, _NLP_N8, _NLP_FULL = 16 * 513, 1026, 256
_IMG_ROWS, _IMG_N8, _IMG_FULL = 16 * 145, 290, 72
_TMP_ROWS, _TMP_N8, _TMP_FULL = 128 * 257, 4112, 1024

_S768, _S256 = 32, 64

_NLP_CHUNKS = [(i * 32, 32) for i in range(8)]
_IMG_CHUNKS = [(0, 32), (32, 32), (64, 8)]
_TMP_CHUNKS = [(i * 64, 64) for i in range(16)]


def _flat_src(remain, rows_per_seq):
    lead = remain.reshape(-1, remain.shape[-1]).astype(np.int32)
    n_seq = lead.shape[0]
    src = np.concatenate([np.zeros((n_seq, 1), np.int32), lead + 1], axis=1)
    src += (np.arange(n_seq, dtype=np.int32) * rows_per_seq)[:, None]
    return src.reshape(-1)


_SRC_IMG = _flat_src(_I_REMAIN, 577)
_SRC_TMP = _flat_src(_T_REMAIN, 513)


def _nlp_src(nlp_remain_idx):
    b = nlp_remain_idx.shape[0]
    src = jnp.concatenate(
        [jnp.zeros((b, 1), jnp.int32), nlp_remain_idx.astype(jnp.int32) + 1], axis=1
    )
    src = src + (jnp.arange(b, dtype=jnp.int32) * 2049)[:, None]
    return src.reshape(-1)


class _Ring:
    def __init__(self, bufs, gsems, wsems):
        self.bufs, self.gsems, self.wsems = bufs, gsems, wsems
        self.gh = [None] * len(bufs)
        self.wh = [None] * len(bufs)
        self.last = None
        self.ptr = 0

    def issue(self, hbm, idxbuf, off, out_ref, out_base, rows):
        s = self.ptr % len(self.bufs)
        self.ptr += 1
        if self.wh[s] is not None:
            self.wh[s].wait()
            self.wh[s] = None
        self.gh[s] = pltpu.async_copy(
            hbm.at[idxbuf.at[pl.ds(off, rows)]],
            self.bufs[s].at[pl.ds(0, rows)],
            self.gsems[s],
        )
        if self.last is not None:
            ls, lout, lbase, lrows = self.last
            self.gh[ls].wait()
            self.gh[ls] = None
            self.wh[ls] = pltpu.async_copy(
                self.bufs[ls].at[pl.ds(0, lrows)],
                lout.at[pl.ds(lbase, lrows)],
                self.wsems[ls],
            )
        self.last = (s, out_ref, out_base, rows)

    def drain(self):
        if self.last is not None:
            ls, lout, lbase, lrows = self.last
            self.gh[ls].wait()
            self.wh[ls] = pltpu.async_copy(
                self.bufs[ls].at[pl.ds(0, lrows)],
                lout.at[pl.ds(lbase, lrows)],
                self.wsems[ls],
            )
            self.last = None
        for s, h in enumerate(self.wh):
            if h is not None:
                h.wait()
                self.wh[s] = None


def _share(wid, n8):
    b = ((wid * n8) // 32) * 8
    e = (((wid + 1) * n8) // 32) * 8
    return b, e


def _gather_body(nlp_hbm, img_hbm, tmp_hbm, src_nlp, src_img, src_tmp,
                 out_nlp, out_img, out_tmp,
                 idx_nlp, idx_img, idx_tmp,
                 ti_nlp, ti_img, ti_tmp,
                 d768a, d768b, d768c, d256a, d256b, d256c,
                 g768a, g768b, g768c, w768a, w768b, w768c,
                 g256a, g256b, g256c, w256a, w256b, w256c,
                 tg_a, tg_b, tg_c, tw_a, tw_b, tw_c):
    wid = lax.axis_index("s") * _NC + lax.axis_index("c")
    b_nlp, e_nlp = _share(wid, _NLP_N8)
    b_img, e_img = _share(wid, _IMG_N8)
    b_tmp, e_tmp = _share(wid, _TMP_N8)

    pltpu.sync_copy(src_img.at[pl.ds(b_img, _IMG_FULL)], idx_img)
    pltpu.sync_copy(src_nlp.at[pl.ds(b_nlp, _NLP_FULL)], idx_nlp)
    pltpu.sync_copy(src_tmp.at[pl.ds(b_tmp, _TMP_FULL)], idx_tmp)

    r768 = _Ring([d768a, d768b, d768c], [g768a, g768b, g768c],
                 [w768a, w768b, w768c])
    r256 = _Ring([d256a, d256b, d256c], [g256a, g256b, g256c],
                 [w256a, w256b, w256c])

    t768 = ([(r768, img_hbm, idx_img, off, out_img, b_img + off, rows)
             for off, rows in _IMG_CHUNKS] +
            [(r768, nlp_hbm, idx_nlp, off, out_nlp, b_nlp + off, rows)
             for off, rows in _NLP_CHUNKS])
    t256 = [(r256, tmp_hbm, idx_tmp, off, out_tmp, b_tmp + off, rows)
            for off, rows in _TMP_CHUNKS]

    merged = []
    n = max(len(t768), len(t256))
    for i in range(n):
        if i < len(t256):
            merged.append(t256[i])
        if i < len(t768):
            merged.append(t768[i])
    for ring, hbm, idxbuf, off, out_ref, out_base, rows in merged:
        ring.issue(hbm, idxbuf, off, out_ref, out_base, rows)

    tails = [
        (nlp_hbm, src_nlp, out_nlp, ti_nlp,
         jnp.where(e_nlp - b_nlp > _NLP_FULL, b_nlp + _NLP_FULL,
                   b_nlp + _NLP_FULL - 8),
         d768a, tg_a, tw_a),
        (img_hbm, src_img, out_img, ti_img,
         jnp.where(e_img - b_img > _IMG_FULL, b_img + _IMG_FULL,
                   b_img + _IMG_FULL - 8),
         d768b, tg_b, tw_b),
        (tmp_hbm, src_tmp, out_tmp, ti_tmp,
         jnp.where(e_tmp - b_tmp > _TMP_FULL, b_tmp + _TMP_FULL,
                   b_tmp + _TMP_FULL - 8),
         d256a, tg_c, tw_c),
    ]
    r768.drain()
    r256.drain()
    gh = []
    for hbm, src, out_ref, tibuf, gbase, dbuf, gsem, wsem in tails:
        pltpu.sync_copy(src.at[pl.ds(gbase, 8)], tibuf)
        gh.append(pltpu.async_copy(hbm.at[tibuf], dbuf.at[pl.ds(0, 8)], gsem))
    wh = []
    for (hbm, src, out_ref, tibuf, gbase, dbuf, gsem, wsem), h in zip(tails, gh):
        h.wait()
        wh.append(pltpu.async_copy(dbuf.at[pl.ds(0, 8)],
                                   out_ref.at[pl.ds(gbase, 8)], wsem))
    for h in wh:
        h.wait()


_gather_call = pl.kernel(
    _gather_body,
    out_type=(
        jax.ShapeDtypeStruct((_NLP_ROWS, 768), jnp.float32),
        jax.ShapeDtypeStruct((_IMG_ROWS, 768), jnp.float32),
        jax.ShapeDtypeStruct((_TMP_ROWS, 256), jnp.float32),
    ),
    mesh=plsc.VectorSubcoreMesh(core_axis_name="c", subcore_axis_name="s"),
    scratch_types=(
        pltpu.VMEM((_NLP_FULL,), jnp.int32),
        pltpu.VMEM((_IMG_FULL,), jnp.int32),
        pltpu.VMEM((_TMP_FULL,), jnp.int32),
        pltpu.VMEM((8,), jnp.int32),
        pltpu.VMEM((8,), jnp.int32),
        pltpu.VMEM((8,), jnp.int32),
        pltpu.VMEM((_S768, 768), jnp.float32),
        pltpu.VMEM((_S768, 768), jnp.float32),
        pltpu.VMEM((_S768, 768), jnp.float32),
        pltpu.VMEM((_S256, 256), jnp.float32),
        pltpu.VMEM((_S256, 256), jnp.float32),
        pltpu.VMEM((_S256, 256), jnp.float32),
    ) + (pltpu.SemaphoreType.DMA,) * 18,
)


def kernel(temporal_block, img, nlp, nlp_remain_idx, nlp_masked_idx,
           nlp_revert_idx, nlp_revert_padding_mask):
    nlp_flat = nlp.reshape(-1, nlp.shape[-1])
    img_flat = img.reshape(-1, img.shape[-1])
    tmp_flat = temporal_block.reshape(-1, temporal_block.shape[-1])
    src_nlp = _nlp_src(nlp_remain_idx)

    out_nlp_p, out_img_p, out_tmp_p = _gather_call(
        nlp_flat, img_flat, tmp_flat,
        src_nlp, jnp.asarray(_SRC_IMG), jnp.asarray(_SRC_TMP),
    )

    temporal_remain_block = out_tmp_p.reshape(_B, 8, 257, 256)
    img_remain = out_img_p.reshape(_B, 145, 768)
    nlp_remain = out_nlp_p.reshape(_B, 513, 768)

    ng_pm = nlp_revert_padding_mask[:, :1]
    nv_pm = nlp_revert_padding_mask[:, 1:]
    nr_pm = jnp.take_along_axis(nv_pm, nlp_remain_idx, axis=1)
    nm_pm = jnp.take_along_axis(nv_pm, nlp_masked_idx, axis=1)
    nlp_remain_pm = jnp.concatenate([ng_pm, nr_pm], axis=1)
    nlp_masked_pm = jnp.concatenate([ng_pm, nm_pm], axis=1)
    img_remain_pm = jnp.ones((_B, 145), jnp.float32)
    img_masked_pm = jnp.ones((_B, 433), jnp.float32)
    img_revert_pm = jnp.ones((_B, 577), jnp.float32)

    return (temporal_remain_block, jnp.asarray(_T_MASKED), jnp.asarray(_T_REVERT),
            img_remain, jnp.asarray(_I_MASKED), jnp.asarray(_I_REVERT),
            img_remain_pm, img_masked_pm, img_revert_pm,
            nlp_remain, nlp_remain_pm, nlp_masked_pm, nlp_revert_padding_mask)


# speedup vs baseline: 1.4124x; 1.0476x over previous
import jax
import jax.numpy as jnp
import numpy as np
from jax import lax
from jax.experimental import pallas as pl
from jax.experimental.pallas import tpu as pltpu
from jax.experimental.pallas import tpu_sc as plsc

_B = 16
_NC, _NS = 2, 16
_NW = _NC * _NS


def _fixed_indices(seed, shape, num_remain):
    with jax.default_device(jax.devices("cpu")[0]):
        noise = jax.random.uniform(jax.random.key(seed), shape)
        shuffle_idx = jnp.argsort(noise, axis=-1)
        remain = np.asarray(shuffle_idx[..., :num_remain], dtype=np.int32)
        masked = np.asarray(shuffle_idx[..., num_remain:], dtype=np.int32)
        revert = np.asarray(jnp.argsort(shuffle_idx, axis=-1), dtype=np.int32)
    return remain, masked, revert


_T_REMAIN, _T_MASKED, _T_REVERT = _fixed_indices(1, (_B, 8, 512), 256)
_I_REMAIN, _I_MASKED, _I_REVERT = _fixed_indices(2, (_B, 576), 144)


def _flat_src(remain, rows_per_seq):
    lead = remain.reshape(-1, remain.shape[-1]).astype(np.int32)
    n_seq = lead.shape[0]
    src = np.concatenate([np.zeros((n_seq, 1), np.int32), lead + 1], axis=1)
    src += (np.arange(n_seq, dtype=np.int32) * rows_per_seq)[:, None]
    return src.reshape(-1)


_SRC_IMG = _flat_src(_I_REMAIN, 577)
_SRC_TMP = _flat_src(_T_REMAIN, 513)


_NLP_W, _IMG_W, _TMP_W = 264, 80, 264


def _img_aligned():
    w = np.arange(32)
    s, half = w // 2, w % 2
    base = s * 145 + half * 72
    pos = np.minimum(base[:, None] + np.arange(_IMG_W)[None, :],
                     (s * 145 + 144)[:, None])
    return _SRC_IMG[pos].reshape(-1).astype(np.int32)


def _tmp_aligned():
    seq = (4 * np.arange(32))[:, None] + np.arange(4)[None, :]
    pos = seq[..., None] * 257 + np.minimum(np.arange(_TMP_W)[None, None, :], 256)
    return _SRC_TMP[pos].reshape(-1).astype(np.int32)


_SRC_IMG_AL = _img_aligned()
_SRC_TMP_AL = _tmp_aligned()


def _nlp_src(nlp_remain_idx):
    b = nlp_remain_idx.shape[0]
    src = jnp.concatenate(
        [jnp.zeros((b, 1), jnp.int32), nlp_remain_idx.astype(jnp.int32) + 1], axis=1
    )
    src = src + (jnp.arange(b, dtype=jnp.int32) * 2049)[:, None]
    return src.reshape(-1)


def _nlp_aligned(src_nlp):
    w = jnp.arange(32)
    s, half = w // 2, w % 2
    base = s * 513 + half * 256
    pos = jnp.minimum(base[:, None] + jnp.arange(_NLP_W)[None, :],
                      (s * 513 + 512)[:, None])
    return jnp.take(src_nlp, pos).reshape(-1)


_NLP_CHUNKS = [(i * 32, 32) for i in range(8)]
_IMG_CHUNKS = [(0, 32), (32, 32), (64, 8)]
_TMP_CHUNKS = [(i * 64, 64) for i in range(4)]

_S768, _S256 = 32, 64


class _Ring:
    def __init__(self, bufs, gsems, wsems):
        self.bufs, self.gsems, self.wsems = bufs, gsems, wsems
        self.gh = [None] * len(bufs)
        self.wh = [None] * len(bufs)
        self.last = None
        self.ptr = 0

    def issue(self, hbm, idxbuf, off, out_ref, out_base, rows):
        s = self.ptr % len(self.bufs)
        self.ptr += 1
        if self.wh[s] is not None:
            self.wh[s].wait()
            self.wh[s] = None
        self.gh[s] = pltpu.async_copy(
            hbm.at[idxbuf.at[pl.ds(off, rows)]],
            self.bufs[s].at[pl.ds(0, rows)],
            self.gsems[s],
        )
        if self.last is not None:
            ls, lout, lbase, lrows = self.last
            self.gh[ls].wait()
            self.gh[ls] = None
            self.wh[ls] = pltpu.async_copy(
                self.bufs[ls].at[pl.ds(0, lrows)],
                lout.at[pl.ds(lbase, lrows)],
                self.wsems[ls],
            )
        self.last = (s, out_ref, out_base, rows)

    def drain(self):
        if self.last is not None:
            ls, lout, lbase, lrows = self.last
            self.gh[ls].wait()
            self.wh[ls] = pltpu.async_copy(
                self.bufs[ls].at[pl.ds(0, lrows)],
                lout.at[pl.ds(lbase, lrows)],
                self.wsems[ls],
            )
            self.last = None
        for s, h in enumerate(self.wh):
            if h is not None:
                h.wait()
                self.wh[s] = None


def _gather_body(nlp_hbm, img_hbm, tmp_hbm, src_nlp, src_img, src_tmp,
                 out_nlp, out_img, out_tmp,
                 idx_nlp, idx_img, idx_tmp,
                 d768a, d768b, d768c, d256a, d256b, d256c,
                 g768a, g768b, g768c, w768a, w768b, w768c,
                 g256a, g256b, g256c, w256a, w256b, w256c,
                 tg, tw):
    wid = lax.axis_index("s") * _NC + lax.axis_index("c")
    s2 = wid // 2
    half = wid % 2
    nlp_base = half * 256
    img_base = half * 72

    pltpu.sync_copy(src_nlp.at[pl.ds(wid * _NLP_W, _NLP_W)], idx_nlp)
    pltpu.sync_copy(src_img.at[pl.ds(wid * _IMG_W, _IMG_W)], idx_img)
    pltpu.sync_copy(src_tmp.at[pl.ds(wid * 4 * _TMP_W, 4 * _TMP_W)], idx_tmp)

    o_nlp = out_nlp.at[s2]
    o_img = out_img.at[s2]

    r768 = _Ring([d768a, d768b, d768c], [g768a, g768b, g768c],
                 [w768a, w768b, w768c])
    r256 = _Ring([d256a, d256b, d256c], [g256a, g256b, g256c],
                 [w256a, w256b, w256c])

    t768 = ([(r768, img_hbm, idx_img, off, o_img, img_base + off, rows)
             for off, rows in _IMG_CHUNKS] +
            [(r768, nlp_hbm, idx_nlp, off, o_nlp, nlp_base + off, rows)
             for off, rows in _NLP_CHUNKS])
    t256 = []
    o_tmp = []
    for j in range(4):
        seq = wid * 4 + j
        o_t = out_tmp.at[seq // 8, seq % 8]
        o_tmp.append(o_t)
        t256 += [(r256, tmp_hbm, idx_tmp, j * _TMP_W + off, o_t, off, rows)
                 for off, rows in _TMP_CHUNKS]

    merged = []
    n = max(len(t768), len(t256))
    for i in range(n):
        if i < len(t256):
            merged.append(t256[i])
        if i < len(t768):
            merged.append(t768[i])
    for ring, hbm, idxbuf, off, out_ref, out_base, rows in merged:
        ring.issue(hbm, idxbuf, off, out_ref, out_base, rows)
    r768.drain()
    r256.drain()

    for j in range(4):
        pltpu.async_copy(
            tmp_hbm.at[idx_tmp.at[pl.ds(j * _TMP_W + 256, 8)]],
            d256a.at[pl.ds(8 * j, 8)], tg).wait()
        pltpu.async_copy(d256a.at[pl.ds(8 * j, 1)],
                         o_tmp[j].at[pl.ds(256, 1)], tw).wait()

    @pl.when(half == 1)
    def _tails_768():
        pltpu.async_copy(nlp_hbm.at[idx_nlp.at[pl.ds(256, 8)]],
                         d768a.at[pl.ds(0, 8)], tg).wait()
        pltpu.async_copy(d768a.at[pl.ds(0, 1)],
                         o_nlp.at[pl.ds(512, 1)], tw).wait()
        pltpu.async_copy(img_hbm.at[idx_img.at[pl.ds(72, 8)]],
                         d768a.at[pl.ds(8, 8)], tg).wait()
        pltpu.async_copy(d768a.at[pl.ds(8, 1)],
                         o_img.at[pl.ds(144, 1)], tw).wait()


_gather_call = pl.kernel(
    _gather_body,
    out_type=(
        jax.ShapeDtypeStruct((_B, 513, 768), jnp.float32),
        jax.ShapeDtypeStruct((_B, 145, 768), jnp.float32),
        jax.ShapeDtypeStruct((_B, 8, 257, 256), jnp.float32),
    ),
    mesh=plsc.VectorSubcoreMesh(core_axis_name="c", subcore_axis_name="s"),
    scratch_types=(
        pltpu.VMEM((_NLP_W,), jnp.int32),
        pltpu.VMEM((_IMG_W,), jnp.int32),
        pltpu.VMEM((4 * _TMP_W,), jnp.int32),
        pltpu.VMEM((_S768, 768), jnp.float32),
        pltpu.VMEM((_S768, 768), jnp.float32),
        pltpu.VMEM((_S768, 768), jnp.float32),
        pltpu.VMEM((_S256, 256), jnp.float32),
        pltpu.VMEM((_S256, 256), jnp.float32),
        pltpu.VMEM((_S256, 256), jnp.float32),
    ) + (pltpu.SemaphoreType.DMA,) * 14,
)


def kernel(temporal_block, img, nlp, nlp_remain_idx, nlp_masked_idx,
           nlp_revert_idx, nlp_revert_padding_mask):
    nlp_flat = nlp.reshape(-1, nlp.shape[-1])
    img_flat = img.reshape(-1, img.shape[-1])
    tmp_flat = temporal_block.reshape(-1, temporal_block.shape[-1])
    src_nlp = _nlp_aligned(_nlp_src(nlp_remain_idx))

    nlp_remain, img_remain, temporal_remain_block = _gather_call(
        nlp_flat, img_flat, tmp_flat,
        src_nlp, jnp.asarray(_SRC_IMG_AL), jnp.asarray(_SRC_TMP_AL),
    )

    ng_pm = nlp_revert_padding_mask[:, :1]
    nv_pm = nlp_revert_padding_mask[:, 1:]
    nr_pm = jnp.take_along_axis(nv_pm, nlp_remain_idx, axis=1)
    nm_pm = jnp.take_along_axis(nv_pm, nlp_masked_idx, axis=1)
    nlp_remain_pm = jnp.concatenate([ng_pm, nr_pm], axis=1)
    nlp_masked_pm = jnp.concatenate([ng_pm, nm_pm], axis=1)
    img_remain_pm = jnp.ones((_B, 145), jnp.float32)
    img_masked_pm = jnp.ones((_B, 433), jnp.float32)
    img_revert_pm = jnp.ones((_B, 577), jnp.float32)

    return (temporal_remain_block, jnp.asarray(_T_MASKED), jnp.asarray(_T_REVERT),
            img_remain, jnp.asarray(_I_MASKED), jnp.asarray(_I_REVERT),
            img_remain_pm, img_masked_pm, img_revert_pm,
            nlp_remain, nlp_remain_pm, nlp_masked_pm, nlp_revert_padding_mask)


# speedup vs baseline: 2.2919x; 1.6227x over previous
import jax
import jax.numpy as jnp
import numpy as np
from jax import lax
from jax.experimental import pallas as pl
from jax.experimental.pallas import tpu as pltpu
from jax.experimental.pallas import tpu_sc as plsc

_B = 16
_NC, _NS = 2, 16
_NW = _NC * _NS


def _fixed_indices(seed, shape, num_remain):
    with jax.default_device(jax.devices("cpu")[0]):
        noise = jax.random.uniform(jax.random.key(seed), shape)
        shuffle_idx = jnp.argsort(noise, axis=-1)
        remain = np.asarray(shuffle_idx[..., :num_remain], dtype=np.int32)
        masked = np.asarray(shuffle_idx[..., num_remain:], dtype=np.int32)
        revert = np.asarray(jnp.argsort(shuffle_idx, axis=-1), dtype=np.int32)
    return remain, masked, revert


_T_REMAIN, _T_MASKED, _T_REVERT = _fixed_indices(1, (_B, 8, 512), 256)
_I_REMAIN, _I_MASKED, _I_REVERT = _fixed_indices(2, (_B, 576), 144)


def _local_src(remain):
    lead = remain.reshape(-1, remain.shape[-1]).astype(np.int32)
    n_seq = lead.shape[0]
    return np.concatenate([np.zeros((n_seq, 1), np.int32), lead + 1], axis=1)


_SRC_IMG = _local_src(_I_REMAIN)
_SRC_TMP = _local_src(_T_REMAIN)


_NLP_W, _IMG_W, _TMP_W = 264, 80, 264


def _img_aligned():
    w = np.arange(32)
    s, half = w // 2, w % 2
    cols = np.minimum(half[:, None] * 72 + np.arange(_IMG_W)[None, :], 144)
    return _SRC_IMG[s[:, None], cols].reshape(-1).astype(np.int32)


def _tmp_aligned():
    seq = (4 * np.arange(32))[:, None] + np.arange(4)[None, :]
    cols = np.minimum(np.arange(_TMP_W), 256)
    return _SRC_TMP[seq[..., None],
                    cols[None, None, :]].reshape(-1).astype(np.int32)


_SRC_IMG_AL = _img_aligned()
_SRC_TMP_AL = _tmp_aligned()


def _nlp_aligned(nlp_remain_idx):
    b = nlp_remain_idx.shape[0]
    src = jnp.concatenate(
        [jnp.zeros((b, 1), jnp.int32), nlp_remain_idx.astype(jnp.int32) + 1], axis=1
    )
    w = jnp.arange(32)
    s, half = w // 2, w % 2
    cols = jnp.minimum(half[:, None] * 256 + jnp.arange(_NLP_W)[None, :], 512)
    return src[s[:, None], cols].reshape(-1)


_NLP_CHUNKS = [(i * 32, 32) for i in range(8)]
_IMG_CHUNKS = [(0, 32), (32, 32), (64, 8)]
_TMP_CHUNKS = [(i * 64, 64) for i in range(4)]

_S768, _S256 = 32, 64


class _Ring:
    def __init__(self, bufs, gsems, wsems):
        self.bufs, self.gsems, self.wsems = bufs, gsems, wsems
        self.gh = [None] * len(bufs)
        self.wh = [None] * len(bufs)
        self.last = None
        self.ptr = 0

    def issue(self, hbm, idxbuf, off, out_ref, out_base, rows):
        s = self.ptr % len(self.bufs)
        self.ptr += 1
        if self.wh[s] is not None:
            self.wh[s].wait()
            self.wh[s] = None
        self.gh[s] = pltpu.async_copy(
            hbm.at[idxbuf.at[pl.ds(off, rows)]],
            self.bufs[s].at[pl.ds(0, rows)],
            self.gsems[s],
        )
        if self.last is not None:
            ls, lout, lbase, lrows = self.last
            self.gh[ls].wait()
            self.gh[ls] = None
            self.wh[ls] = pltpu.async_copy(
                self.bufs[ls].at[pl.ds(0, lrows)],
                lout.at[pl.ds(lbase, lrows)],
                self.wsems[ls],
            )
        self.last = (s, out_ref, out_base, rows)

    def drain(self):
        if self.last is not None:
            ls, lout, lbase, lrows = self.last
            self.gh[ls].wait()
            self.wh[ls] = pltpu.async_copy(
                self.bufs[ls].at[pl.ds(0, lrows)],
                lout.at[pl.ds(lbase, lrows)],
                self.wsems[ls],
            )
            self.last = None
        for s, h in enumerate(self.wh):
            if h is not None:
                h.wait()
                self.wh[s] = None


def _gather_body(nlp_hbm, img_hbm, tmp_hbm, src_nlp, src_img, src_tmp,
                 out_nlp, out_img, out_tmp,
                 idx_nlp, idx_img, idx_tmp,
                 d768a, d768b, d768c, d256a, d256b, d256c,
                 g768a, g768b, g768c, w768a, w768b, w768c,
                 g256a, g256b, g256c, w256a, w256b, w256c,
                 tg, tw):
    wid = lax.axis_index("s") * _NC + lax.axis_index("c")
    s2 = wid // 2
    half = wid % 2
    nlp_base = half * 256
    img_base = half * 72

    pltpu.sync_copy(src_nlp.at[pl.ds(wid * _NLP_W, _NLP_W)], idx_nlp)
    pltpu.sync_copy(src_img.at[pl.ds(wid * _IMG_W, _IMG_W)], idx_img)
    pltpu.sync_copy(src_tmp.at[pl.ds(wid * 4 * _TMP_W, 4 * _TMP_W)], idx_tmp)

    o_nlp = out_nlp.at[s2]
    o_img = out_img.at[s2]
    h_nlp = nlp_hbm.at[s2]
    h_img = img_hbm.at[s2]

    r768 = _Ring([d768a, d768b, d768c], [g768a, g768b, g768c],
                 [w768a, w768b, w768c])
    r256 = _Ring([d256a, d256b, d256c], [g256a, g256b, g256c],
                 [w256a, w256b, w256c])

    t768 = ([(r768, h_img, idx_img, off, o_img, img_base + off, rows)
             for off, rows in _IMG_CHUNKS] +
            [(r768, h_nlp, idx_nlp, off, o_nlp, nlp_base + off, rows)
             for off, rows in _NLP_CHUNKS])
    t256 = []
    o_tmp = []
    h_tmp = []
    for j in range(4):
        seq = wid * 4 + j
        o_t = out_tmp.at[seq // 8, seq % 8]
        h_t = tmp_hbm.at[seq // 8, seq % 8]
        o_tmp.append(o_t)
        h_tmp.append(h_t)
        t256 += [(r256, h_t, idx_tmp, j * _TMP_W + off, o_t, off, rows)
                 for off, rows in _TMP_CHUNKS]

    merged = []
    n = max(len(t768), len(t256))
    for i in range(n):
        if i < len(t256):
            merged.append(t256[i])
        if i < len(t768):
            merged.append(t768[i])
    for ring, hbm, idxbuf, off, out_ref, out_base, rows in merged:
        ring.issue(hbm, idxbuf, off, out_ref, out_base, rows)
    r768.drain()
    r256.drain()

    for j in range(4):
        pltpu.async_copy(
            h_tmp[j].at[idx_tmp.at[pl.ds(j * _TMP_W + 256, 8)]],
            d256a.at[pl.ds(8 * j, 8)], tg).wait()
        pltpu.async_copy(d256a.at[pl.ds(8 * j, 1)],
                         o_tmp[j].at[pl.ds(256, 1)], tw).wait()

    @pl.when(half == 1)
    def _tails_768():
        pltpu.async_copy(h_nlp.at[idx_nlp.at[pl.ds(256, 8)]],
                         d768a.at[pl.ds(0, 8)], tg).wait()
        pltpu.async_copy(d768a.at[pl.ds(0, 1)],
                         o_nlp.at[pl.ds(512, 1)], tw).wait()
        pltpu.async_copy(h_img.at[idx_img.at[pl.ds(72, 8)]],
                         d768a.at[pl.ds(8, 8)], tg).wait()
        pltpu.async_copy(d768a.at[pl.ds(8, 1)],
                         o_img.at[pl.ds(144, 1)], tw).wait()


_gather_call = pl.kernel(
    _gather_body,
    out_type=(
        jax.ShapeDtypeStruct((_B, 513, 768), jnp.float32),
        jax.ShapeDtypeStruct((_B, 145, 768), jnp.float32),
        jax.ShapeDtypeStruct((_B, 8, 257, 256), jnp.float32),
    ),
    mesh=plsc.VectorSubcoreMesh(core_axis_name="c", subcore_axis_name="s"),
    scratch_types=(
        pltpu.VMEM((_NLP_W,), jnp.int32),
        pltpu.VMEM((_IMG_W,), jnp.int32),
        pltpu.VMEM((4 * _TMP_W,), jnp.int32),
        pltpu.VMEM((_S768, 768), jnp.float32),
        pltpu.VMEM((_S768, 768), jnp.float32),
        pltpu.VMEM((_S768, 768), jnp.float32),
        pltpu.VMEM((_S256, 256), jnp.float32),
        pltpu.VMEM((_S256, 256), jnp.float32),
        pltpu.VMEM((_S256, 256), jnp.float32),
    ) + (pltpu.SemaphoreType.DMA,) * 14,
)


def kernel(temporal_block, img, nlp, nlp_remain_idx, nlp_masked_idx,
           nlp_revert_idx, nlp_revert_padding_mask):
    nlp_remain, img_remain, temporal_remain_block = _gather_call(
        nlp, img, temporal_block,
        _nlp_aligned(nlp_remain_idx),
        jnp.asarray(_SRC_IMG_AL), jnp.asarray(_SRC_TMP_AL),
    )

    ng_pm = nlp_revert_padding_mask[:, :1]
    nv_pm = nlp_revert_padding_mask[:, 1:]
    nr_pm = jnp.take_along_axis(nv_pm, nlp_remain_idx, axis=1)
    nm_pm = jnp.take_along_axis(nv_pm, nlp_masked_idx, axis=1)
    nlp_remain_pm = jnp.concatenate([ng_pm, nr_pm], axis=1)
    nlp_masked_pm = jnp.concatenate([ng_pm, nm_pm], axis=1)
    img_remain_pm = jnp.ones((_B, 145), jnp.float32)
    img_masked_pm = jnp.ones((_B, 433), jnp.float32)
    img_revert_pm = jnp.ones((_B, 577), jnp.float32)

    return (temporal_remain_block, jnp.asarray(_T_MASKED), jnp.asarray(_T_REVERT),
            img_remain, jnp.asarray(_I_MASKED), jnp.asarray(_I_REVERT),
            img_remain_pm, img_masked_pm, img_revert_pm,
            nlp_remain, nlp_remain_pm, nlp_masked_pm, nlp_revert_padding_mask)
